# Initial kernel scaffold; baseline (speedup 1.0000x reference)
import functools
import jax
import jax.numpy as jnp
from jax import lax
from jax.experimental import pallas as pl
from jax.experimental.pallas import tpu as pltpu

N = 2000; E = 16000; B = 8; D = 1024; C = 128; S = 7
H = 256; P = 50; K = 10; TOPX = 100
NPG = N // B; EPG = E // B
SS = S * S
EB = 1000
TW = 16

_INTERPRET = False


def _conv_stats_body(cv_ref, pooled_ref, pos_ref, neg_ref):
    x = cv_ref[...]
    r = jnp.maximum(x, 0.0)
    pooled_ref[...] = jnp.sum(x, axis=2) * (1.0 / SS)
    pos_ref[...] = jnp.sum(r, axis=2)
    neg_ref[...] = jnp.sum(r - x, axis=2)


def _conv_stats(cv3):
    BN = 200
    grid = (N // BN,)
    spec_in = pl.BlockSpec((BN, C, SS), lambda i: (i, 0, 0))
    spec_out = pl.BlockSpec((BN, C), lambda i: (i, 0))
    return pl.pallas_call(
        _conv_stats_body, grid=grid,
        in_specs=[spec_in],
        out_specs=[spec_out, spec_out, spec_out],
        out_shape=[jax.ShapeDtypeStruct((N, C), jnp.float32)] * 3,
        interpret=_INTERPRET,
    )(cv3)


def _node_fwd_body(ol_ref, pooled_ref, W1_ref, W2_ref, node_ref, gnode_ref):
    pre = (jnp.dot(ol_ref[0], W1_ref[...], preferred_element_type=jnp.float32)
           + jnp.dot(pooled_ref[0], W2_ref[...], preferred_element_type=jnp.float32))
    nr = jnp.maximum(pre, 0.0)
    node_ref[0] = nr
    gnode_ref[0, 0] = jnp.sum(nr, axis=0)


def _node_fwd(ol3, pooled3, W1, W2):
    return pl.pallas_call(
        _node_fwd_body, grid=(B,),
        in_specs=[
            pl.BlockSpec((1, NPG, D), lambda b: (b, 0, 0)),
            pl.BlockSpec((1, NPG, C), lambda b: (b, 0, 0)),
            pl.BlockSpec((D, H), lambda b: (0, 0)),
            pl.BlockSpec((C, H), lambda b: (0, 0)),
        ],
        out_specs=[
            pl.BlockSpec((1, NPG, H), lambda b: (b, 0, 0)),
            pl.BlockSpec((1, 1, H), lambda b: (b, 0, 0)),
        ],
        out_shape=[
            jax.ShapeDtypeStruct((B, NPG, H), jnp.float32),
            jax.ShapeDtypeStruct((B, 1, H), jnp.float32),
        ],
        interpret=_INTERPRET,
    )(ol3, pooled3, W1, W2)


def _edge_fwd_body(rl_ref, W3_ref, edge_ref, gedge_ref):
    j = pl.program_id(1)
    pre = jnp.dot(rl_ref[0], W3_ref[...], preferred_element_type=jnp.float32)
    nr = jnp.maximum(pre, 0.0)
    edge_ref[0] = nr
    s = jnp.sum(nr, axis=0)

    @pl.when(j == 0)
    def _():
        gedge_ref[0, 0] = s

    @pl.when(j > 0)
    def _():
        gedge_ref[0, 0] += s


def _edge_fwd(rl3, W3):
    return pl.pallas_call(
        _edge_fwd_body, grid=(B, EPG // EB),
        in_specs=[
            pl.BlockSpec((1, EB, D), lambda b, j: (b, j, 0)),
            pl.BlockSpec((D, H), lambda b, j: (0, 0)),
        ],
        out_specs=[
            pl.BlockSpec((1, EB, H), lambda b, j: (b, j, 0)),
            pl.BlockSpec((1, 1, H), lambda b, j: (b, 0, 0)),
        ],
        out_shape=[
            jax.ShapeDtypeStruct((B, EPG, H), jnp.float32),
            jax.ShapeDtypeStruct((B, 1, H), jnp.float32),
        ],
        interpret=_INTERPRET,
    )(rl3, W3)


def _head_body(gn_ref, ge_ref, Wout_ref, WoutT_ref,
               ps_ref, cls_ref, u_ref, ue_ref):
    g = gn_ref[:, 0, :] + ge_ref[:, 0, :]
    logits = jnp.dot(g, Wout_ref[...], preferred_element_type=jnp.float32)
    probs = jax.nn.sigmoid(logits)
    iota = lax.broadcasted_iota(jnp.int32, (B, P), 1)
    v = probs
    vals, clss = [], []
    for _ in range(K):
        m = jnp.max(v, axis=1, keepdims=True)
        cls = jnp.min(jnp.where(v >= m, iota, P), axis=1, keepdims=True)
        vals.append(m)
        clss.append(cls)
        v = jnp.where(iota == cls, -1.0, v)
    ps = jnp.concatenate(vals, axis=1)
    ps_ref[...] = ps
    cls_ref[...] = jnp.concatenate(clss, axis=1)
    us, ues = [], []
    for k in range(K):
        onehot = (iota == clss[k]).astype(jnp.float32)
        sp = vals[k] * (1.0 - vals[k])
        u_k = jnp.dot(onehot * sp, WoutT_ref[...],
                      preferred_element_type=jnp.float32)
        us.append(u_k[:, None, :])
        ues.append((u_k * vals[k])[:, None, :])
    u_ref[...] = jnp.concatenate(us, axis=1)
    ue_ref[...] = jnp.concatenate(ues, axis=1)


def _head(gnode, gedge, Wout, WoutT):
    return pl.pallas_call(
        _head_body,
        in_specs=[
            pl.BlockSpec((B, 1, H), lambda: (0, 0, 0)),
            pl.BlockSpec((B, 1, H), lambda: (0, 0, 0)),
            pl.BlockSpec((H, P), lambda: (0, 0)),
            pl.BlockSpec((P, H), lambda: (0, 0)),
        ],
        out_specs=[
            pl.BlockSpec((B, K), lambda: (0, 0)),
            pl.BlockSpec((B, K), lambda: (0, 0)),
            pl.BlockSpec((B, K, H), lambda: (0, 0, 0)),
            pl.BlockSpec((B, K, H), lambda: (0, 0, 0)),
        ],
        out_shape=[
            jax.ShapeDtypeStruct((B, K), jnp.float32),
            jax.ShapeDtypeStruct((B, K), jnp.int32),
            jax.ShapeDtypeStruct((B, K, H), jnp.float32),
            jax.ShapeDtypeStruct((B, K, H), jnp.float32),
        ],
        interpret=_INTERPRET,
    )(gnode, gedge, Wout, WoutT)


def _node_rel_body(ol_ref, node_ref, pos_ref, neg_ref, os_ref, u_ref,
                   W1T_ref, W2T_ref, table_ref):
    mask = (node_ref[0] > 0.0).astype(jnp.float32)
    ol = ol_ref[0]
    pos = pos_ref[0]
    neg = neg_ref[0]
    lane = lax.broadcasted_iota(jnp.int32, (NPG, TW), 1)
    tbl = jnp.zeros((NPG, TW), jnp.float32)
    for k in range(K):
        mk = mask * u_ref[0, k][None, :]
        gol = jnp.dot(mk, W1T_ref[...], preferred_element_type=jnp.float32)
        rel_lin = jnp.sum(jnp.maximum(ol * gol, 0.0), axis=1)
        gp = jnp.dot(mk, W2T_ref[...], preferred_element_type=jnp.float32) * (1.0 / SS)
        rel_conv = jnp.sum(jnp.maximum(gp, 0.0) * pos
                           + jnp.maximum(-gp, 0.0) * neg, axis=1)
        tbl = jnp.where(lane == k, (rel_lin + rel_conv)[:, None], tbl)
    tbl = jnp.where(lane == K, os_ref[0, 0][:, None], tbl)
    table_ref[0] = tbl


def _node_rel(ol3, node_relu, pos3, neg3, os3, u, W1T, W2T):
    return pl.pallas_call(
        _node_rel_body, grid=(B,),
        in_specs=[
            pl.BlockSpec((1, NPG, D), lambda b: (b, 0, 0)),
            pl.BlockSpec((1, NPG, H), lambda b: (b, 0, 0)),
            pl.BlockSpec((1, NPG, C), lambda b: (b, 0, 0)),
            pl.BlockSpec((1, NPG, C), lambda b: (b, 0, 0)),
            pl.BlockSpec((1, 1, NPG), lambda b: (b, 0, 0)),
            pl.BlockSpec((1, K, H), lambda b: (b, 0, 0)),
            pl.BlockSpec((H, D), lambda b: (0, 0)),
            pl.BlockSpec((H, C), lambda b: (0, 0)),
        ],
        out_specs=[pl.BlockSpec((1, NPG, TW), lambda b: (b, 0, 0))],
        out_shape=[jax.ShapeDtypeStruct((B, NPG, TW), jnp.float32)],
        interpret=_INTERPRET,
    )(ol3, node_relu, pos3, neg3, os3, u, W1T, W2T)


def _edge_rel_body(rl_ref, edge_ref, ue_ref, W3T_ref, re_ref):
    mask = (edge_ref[0] > 0.0).astype(jnp.float32)
    rl = rl_ref[0]
    lane = lax.broadcasted_iota(jnp.int32, (EB, K), 1)
    re = jnp.zeros((EB, K), jnp.float32)
    for k in range(K):
        mk = mask * ue_ref[0, k][None, :]
        grl = jnp.dot(mk, W3T_ref[...], preferred_element_type=jnp.float32)
        re_k = jnp.sum(jnp.maximum(rl * grl, 0.0), axis=1)
        re = jnp.where(lane == k, re_k[:, None], re)
    re_ref[0] = re


def _edge_rel(rl3, edge_relu, ue, W3T):
    return pl.pallas_call(
        _edge_rel_body, grid=(B, EPG // EB),
        in_specs=[
            pl.BlockSpec((1, EB, D), lambda b, j: (b, j, 0)),
            pl.BlockSpec((1, EB, H), lambda b, j: (b, j, 0)),
            pl.BlockSpec((1, K, H), lambda b, j: (b, 0, 0)),
            pl.BlockSpec((H, D), lambda b, j: (0, 0)),
        ],
        out_specs=[pl.BlockSpec((1, EB, K), lambda b, j: (b, j, 0))],
        out_shape=[jax.ShapeDtypeStruct((B, EPG, K), jnp.float32)],
        interpret=_INTERPRET,
    )(rl3, edge_relu, ue, W3T)


def _score_body(src_ref, dst_ref, table_ref, re_ref, score_ref):
    b = pl.program_id(0)
    srcl = src_ref[0, 0] - b * NPG
    dstl = dst_ref[0, 0] - b * NPG
    nio = lax.broadcasted_iota(jnp.int32, (EPG, NPG), 1)
    oh_s = (srcl[:, None] == nio).astype(jnp.float32)
    oh_d = (dstl[:, None] == nio).astype(jnp.float32)
    tbl = table_ref[0]
    gs = jnp.dot(oh_s, tbl, preferred_element_type=jnp.float32)
    gd = jnp.dot(oh_d, tbl, preferred_element_type=jnp.float32)
    lane = lax.broadcasted_iota(jnp.int32, (EPG, TW), 1)
    osS = jnp.sum(jnp.where(lane == K, gs, 0.0), axis=1, keepdims=True)
    osD = jnp.sum(jnp.where(lane == K, gd, 0.0), axis=1, keepdims=True)
    prod = gs[:, :K] * gd[:, :K] * re_ref[0] * osS * osD
    score_ref[0] = prod


def _score(src3, dst3, table, re):
    return pl.pallas_call(
        _score_body, grid=(B,),
        in_specs=[
            pl.BlockSpec((1, 1, EPG), lambda b: (b, 0, 0)),
            pl.BlockSpec((1, 1, EPG), lambda b: (b, 0, 0)),
            pl.BlockSpec((1, NPG, TW), lambda b: (b, 0, 0)),
            pl.BlockSpec((1, EPG, K), lambda b: (b, 0, 0)),
        ],
        out_specs=[pl.BlockSpec((1, EPG, K), lambda b: (b, 0, 0))],
        out_shape=[jax.ShapeDtypeStruct((B, EPG, K), jnp.float32)],
        interpret=_INTERPRET,
    )(src3, dst3, table, re)


M = EPG * K


def _topk_body(score_ref, vals_ref, idxs_ref, scratch):
    scratch[...] = score_ref[...]
    iota = lax.broadcasted_iota(jnp.int32, (B, M), 1)
    r128 = lax.broadcasted_iota(jnp.int32, (B, 128), 1)

    def body(r, carry):
        vals, idxs = carry
        v = scratch[...]
        m = jnp.max(v, axis=1, keepdims=True)
        f = jnp.min(jnp.where(v >= m, iota, M), axis=1, keepdims=True)
        scratch[...] = jnp.where(iota == f, -jnp.inf, v)
        vals = jnp.where(r128 == r, m, vals)
        idxs = jnp.where(r128 == r, f, idxs)
        return vals, idxs

    vals, idxs = lax.fori_loop(
        0, TOPX, body,
        (jnp.zeros((B, 128), jnp.float32), jnp.zeros((B, 128), jnp.int32)))
    vals_ref[...] = vals
    idxs_ref[...] = idxs


def _topk(score_flat):
    return pl.pallas_call(
        _topk_body,
        in_specs=[pl.BlockSpec((B, M), lambda: (0, 0))],
        out_specs=[pl.BlockSpec((B, 128), lambda: (0, 0)),
                   pl.BlockSpec((B, 128), lambda: (0, 0))],
        out_shape=[jax.ShapeDtypeStruct((B, 128), jnp.float32),
                   jax.ShapeDtypeStruct((B, 128), jnp.int32)],
        scratch_shapes=[pltpu.VMEM((B, M), jnp.float32)],
        interpret=_INTERPRET,
    )(score_flat)


def _assemble_body(idx_ref, sd_ref, ps_ref, clsf_ref,
                   src_ref, dst_ref, psel_ref, csel_ref):
    f = idx_ref[0]
    rel = f // K
    pred = f - rel * K
    relb = jax.lax.broadcast_in_dim(rel, (EPG, 128), (0, 1))
    eio = lax.broadcasted_iota(jnp.int32, (EPG, 128), 0)
    ohT = (eio == relb).astype(jnp.float32)
    src_sel = jnp.dot(sd_ref[0, 0:1, :], ohT, preferred_element_type=jnp.float32)
    dst_sel = jnp.dot(sd_ref[0, 1:2, :], ohT, preferred_element_type=jnp.float32)
    predb = jax.lax.broadcast_in_dim(pred, (K, 128), (0, 1))
    kio = lax.broadcasted_iota(jnp.int32, (K, 128), 0)
    ohP = (kio == predb).astype(jnp.float32)
    psel = jnp.dot(ps_ref[0], ohP, preferred_element_type=jnp.float32)
    csel = jnp.dot(clsf_ref[0], ohP, preferred_element_type=jnp.float32)
    src_ref[0] = src_sel.astype(jnp.int32)
    dst_ref[0] = dst_sel.astype(jnp.int32)
    psel_ref[0] = psel
    csel_ref[0] = csel.astype(jnp.int32)


def _assemble(idxs3, sdf3, ps3, clsf3):
    return pl.pallas_call(
        _assemble_body, grid=(B,),
        in_specs=[
            pl.BlockSpec((1, 1, 128), lambda b: (b, 0, 0)),
            pl.BlockSpec((1, 2, EPG), lambda b: (b, 0, 0)),
            pl.BlockSpec((1, 1, K), lambda b: (b, 0, 0)),
            pl.BlockSpec((1, 1, K), lambda b: (b, 0, 0)),
        ],
        out_specs=[pl.BlockSpec((1, 1, 128), lambda b: (b, 0, 0))] * 4,
        out_shape=[
            jax.ShapeDtypeStruct((B, 1, 128), jnp.int32),
            jax.ShapeDtypeStruct((B, 1, 128), jnp.int32),
            jax.ShapeDtypeStruct((B, 1, 128), jnp.float32),
            jax.ShapeDtypeStruct((B, 1, 128), jnp.int32),
        ],
        interpret=_INTERPRET,
    )(idxs3, sdf3, ps3, clsf3)


def kernel(object_linear_features, object_conv_features, relation_linear_features,
           object_scores, W1, W2, W3, Wout, batch, relation_indexes):
    del batch
    cv3 = object_conv_features.reshape(N, C, SS)
    ol3 = object_linear_features.reshape(B, NPG, D)
    rl3 = relation_linear_features.reshape(B, EPG, D)
    os3 = object_scores.reshape(B, 1, NPG)
    src3 = relation_indexes[0].reshape(B, 1, EPG)
    dst3 = relation_indexes[1].reshape(B, 1, EPG)
    W1T = jnp.transpose(W1)
    W2T = jnp.transpose(W2)
    W3T = jnp.transpose(W3)
    WoutT = jnp.transpose(Wout)

    pooled, pos, neg = _conv_stats(cv3)
    pooled3 = pooled.reshape(B, NPG, C)
    pos3 = pos.reshape(B, NPG, C)
    neg3 = neg.reshape(B, NPG, C)

    node_relu, gnode = _node_fwd(ol3, pooled3, W1, W2)
    edge_relu, gedge = _edge_fwd(rl3, W3)
    ps, cls, u, ue = _head(gnode, gedge, Wout, WoutT)

    table = _node_rel(ol3, node_relu, pos3, neg3, os3, u, W1T, W2T)
    re = _edge_rel(rl3, edge_relu, ue, W3T)
    score = _score(src3, dst3, table, re)

    vals, idxs = _topk(score.reshape(B, M))

    sdf3 = jnp.stack([src3[:, 0, :], dst3[:, 0, :]], axis=1).astype(jnp.float32)
    ps3 = ps.reshape(B, 1, K)
    clsf3 = cls.astype(jnp.float32).reshape(B, 1, K)
    src_sel, dst_sel, psel, csel = _assemble(idxs.reshape(B, 1, 128),
                                             sdf3, ps3, clsf3)

    relation_scores_sorted = vals[:, :TOPX].reshape(-1)
    relation_indexes_sorted = jnp.stack(
        [src_sel[:, 0, :TOPX].reshape(-1), dst_sel[:, 0, :TOPX].reshape(-1)], axis=0)
    predicate_scores_out = psel[:, 0, :TOPX].reshape(-1)
    predicate_classes_out = csel[:, 0, :TOPX].reshape(-1)
    n_relations = jnp.full((B,), TOPX, dtype=jnp.int32)
    return (relation_scores_sorted, relation_indexes_sorted, predicate_scores_out,
            predicate_classes_out, n_relations)


# speedup vs baseline: 3.1151x; 3.1151x over previous
import functools
import jax
import jax.numpy as jnp
from jax import lax
from jax.experimental import pallas as pl
from jax.experimental.pallas import tpu as pltpu

N = 2000; E = 16000; B = 8; D = 1024; C = 128; S = 7
H = 256; P = 50; K = 10; TOPX = 100
NPG = N // B; EPG = E // B
SS = S * S
EB = 1000
TW = 16

_INTERPRET = False


def _conv_stats_body(cv_ref, pooled_ref, pos_ref, neg_ref):
    x = cv_ref[...]
    r = jnp.maximum(x, 0.0)
    pooled_ref[...] = jnp.sum(x, axis=2) * (1.0 / SS)
    pos_ref[...] = jnp.sum(r, axis=2)
    neg_ref[...] = jnp.sum(r - x, axis=2)


def _conv_stats(cv3):
    BN = 200
    grid = (N // BN,)
    spec_in = pl.BlockSpec((BN, C, SS), lambda i: (i, 0, 0))
    spec_out = pl.BlockSpec((BN, C), lambda i: (i, 0))
    return pl.pallas_call(
        _conv_stats_body, grid=grid,
        in_specs=[spec_in],
        out_specs=[spec_out, spec_out, spec_out],
        out_shape=[jax.ShapeDtypeStruct((N, C), jnp.float32)] * 3,
        interpret=_INTERPRET,
    )(cv3)


def _node_fwd_body(ol_ref, pooled_ref, W1_ref, W2_ref, node_ref, gnode_ref):
    pre = (jnp.dot(ol_ref[0], W1_ref[...], preferred_element_type=jnp.float32)
           + jnp.dot(pooled_ref[0], W2_ref[...], preferred_element_type=jnp.float32))
    nr = jnp.maximum(pre, 0.0)
    node_ref[0] = nr
    gnode_ref[0, 0] = jnp.sum(nr, axis=0)


def _node_fwd(ol3, pooled3, W1, W2):
    return pl.pallas_call(
        _node_fwd_body, grid=(B,),
        in_specs=[
            pl.BlockSpec((1, NPG, D), lambda b: (b, 0, 0)),
            pl.BlockSpec((1, NPG, C), lambda b: (b, 0, 0)),
            pl.BlockSpec((D, H), lambda b: (0, 0)),
            pl.BlockSpec((C, H), lambda b: (0, 0)),
        ],
        out_specs=[
            pl.BlockSpec((1, NPG, H), lambda b: (b, 0, 0)),
            pl.BlockSpec((1, 1, H), lambda b: (b, 0, 0)),
        ],
        out_shape=[
            jax.ShapeDtypeStruct((B, NPG, H), jnp.float32),
            jax.ShapeDtypeStruct((B, 1, H), jnp.float32),
        ],
        interpret=_INTERPRET,
    )(ol3, pooled3, W1, W2)


def _edge_fwd_body(rl_ref, W3_ref, edge_ref, gedge_ref):
    j = pl.program_id(1)
    pre = jnp.dot(rl_ref[0], W3_ref[...], preferred_element_type=jnp.float32)
    nr = jnp.maximum(pre, 0.0)
    edge_ref[0] = nr
    s = jnp.sum(nr, axis=0)

    @pl.when(j == 0)
    def _():
        gedge_ref[0, 0] = s

    @pl.when(j > 0)
    def _():
        gedge_ref[0, 0] += s


def _edge_fwd(rl3, W3):
    return pl.pallas_call(
        _edge_fwd_body, grid=(B, EPG // EB),
        in_specs=[
            pl.BlockSpec((1, EB, D), lambda b, j: (b, j, 0)),
            pl.BlockSpec((D, H), lambda b, j: (0, 0)),
        ],
        out_specs=[
            pl.BlockSpec((1, EB, H), lambda b, j: (b, j, 0)),
            pl.BlockSpec((1, 1, H), lambda b, j: (b, 0, 0)),
        ],
        out_shape=[
            jax.ShapeDtypeStruct((B, EPG, H), jnp.float32),
            jax.ShapeDtypeStruct((B, 1, H), jnp.float32),
        ],
        interpret=_INTERPRET,
    )(rl3, W3)


def _head_body(gn_ref, ge_ref, Wout_ref, WoutT_ref,
               ps_ref, cls_ref, u_ref, ue_ref):
    g = gn_ref[:, 0, :] + ge_ref[:, 0, :]
    logits = jnp.dot(g, Wout_ref[...], preferred_element_type=jnp.float32)
    probs = jax.nn.sigmoid(logits)
    iota = lax.broadcasted_iota(jnp.int32, (B, P), 1)
    v = probs
    vals, clss = [], []
    for _ in range(K):
        m = jnp.max(v, axis=1, keepdims=True)
        cls = jnp.min(jnp.where(v >= m, iota, P), axis=1, keepdims=True)
        vals.append(m)
        clss.append(cls)
        v = jnp.where(iota == cls, -1.0, v)
    ps = jnp.concatenate(vals, axis=1)
    ps_ref[...] = ps
    cls_ref[...] = jnp.concatenate(clss, axis=1)
    us, ues = [], []
    for k in range(K):
        onehot = (iota == clss[k]).astype(jnp.float32)
        sp = vals[k] * (1.0 - vals[k])
        u_k = jnp.dot(onehot * sp, WoutT_ref[...],
                      preferred_element_type=jnp.float32)
        us.append(u_k[:, None, :])
        ues.append((u_k * vals[k])[:, None, :])
    u_ref[...] = jnp.concatenate(us, axis=1)
    ue_ref[...] = jnp.concatenate(ues, axis=1)


def _head(gnode, gedge, Wout, WoutT):
    return pl.pallas_call(
        _head_body,
        in_specs=[
            pl.BlockSpec((B, 1, H), lambda: (0, 0, 0)),
            pl.BlockSpec((B, 1, H), lambda: (0, 0, 0)),
            pl.BlockSpec((H, P), lambda: (0, 0)),
            pl.BlockSpec((P, H), lambda: (0, 0)),
        ],
        out_specs=[
            pl.BlockSpec((B, K), lambda: (0, 0)),
            pl.BlockSpec((B, K), lambda: (0, 0)),
            pl.BlockSpec((B, K, H), lambda: (0, 0, 0)),
            pl.BlockSpec((B, K, H), lambda: (0, 0, 0)),
        ],
        out_shape=[
            jax.ShapeDtypeStruct((B, K), jnp.float32),
            jax.ShapeDtypeStruct((B, K), jnp.int32),
            jax.ShapeDtypeStruct((B, K, H), jnp.float32),
            jax.ShapeDtypeStruct((B, K, H), jnp.float32),
        ],
        interpret=_INTERPRET,
    )(gnode, gedge, Wout, WoutT)


def _node_rel_body(ol_ref, node_ref, pos_ref, neg_ref, os_ref, u_ref,
                   W1T_ref, W2T_ref, table_ref):
    mask = (node_ref[0] > 0.0).astype(jnp.float32)
    ol = ol_ref[0]
    pos = pos_ref[0]
    neg = neg_ref[0]
    lane = lax.broadcasted_iota(jnp.int32, (NPG, TW), 1)
    tbl = jnp.zeros((NPG, TW), jnp.float32)
    for k in range(K):
        mk = mask * u_ref[0, k][None, :]
        gol = jnp.dot(mk, W1T_ref[...], preferred_element_type=jnp.float32)
        rel_lin = jnp.sum(jnp.maximum(ol * gol, 0.0), axis=1)
        gp = jnp.dot(mk, W2T_ref[...], preferred_element_type=jnp.float32) * (1.0 / SS)
        rel_conv = jnp.sum(jnp.maximum(gp, 0.0) * pos
                           + jnp.maximum(-gp, 0.0) * neg, axis=1)
        tbl = jnp.where(lane == k, (rel_lin + rel_conv)[:, None], tbl)
    tbl = jnp.where(lane == K, os_ref[0, 0][:, None], tbl)
    table_ref[0] = tbl


def _node_rel(ol3, node_relu, pos3, neg3, os3, u, W1T, W2T):
    return pl.pallas_call(
        _node_rel_body, grid=(B,),
        in_specs=[
            pl.BlockSpec((1, NPG, D), lambda b: (b, 0, 0)),
            pl.BlockSpec((1, NPG, H), lambda b: (b, 0, 0)),
            pl.BlockSpec((1, NPG, C), lambda b: (b, 0, 0)),
            pl.BlockSpec((1, NPG, C), lambda b: (b, 0, 0)),
            pl.BlockSpec((1, 1, NPG), lambda b: (b, 0, 0)),
            pl.BlockSpec((1, K, H), lambda b: (b, 0, 0)),
            pl.BlockSpec((H, D), lambda b: (0, 0)),
            pl.BlockSpec((H, C), lambda b: (0, 0)),
        ],
        out_specs=[pl.BlockSpec((1, NPG, TW), lambda b: (b, 0, 0))],
        out_shape=[jax.ShapeDtypeStruct((B, NPG, TW), jnp.float32)],
        interpret=_INTERPRET,
    )(ol3, node_relu, pos3, neg3, os3, u, W1T, W2T)[0]


def _edge_rel_body(rl_ref, edge_ref, ue_ref, W3T_ref, re_ref):
    mask = (edge_ref[0] > 0.0).astype(jnp.float32)
    rl = rl_ref[0]
    lane = lax.broadcasted_iota(jnp.int32, (EB, K), 1)
    re = jnp.zeros((EB, K), jnp.float32)
    for k in range(K):
        mk = mask * ue_ref[0, k][None, :]
        grl = jnp.dot(mk, W3T_ref[...], preferred_element_type=jnp.float32)
        re_k = jnp.sum(jnp.maximum(rl * grl, 0.0), axis=1)
        re = jnp.where(lane == k, re_k[:, None], re)
    re_ref[0] = re


def _edge_rel(rl3, edge_relu, ue, W3T):
    return pl.pallas_call(
        _edge_rel_body, grid=(B, EPG // EB),
        in_specs=[
            pl.BlockSpec((1, EB, D), lambda b, j: (b, j, 0)),
            pl.BlockSpec((1, EB, H), lambda b, j: (b, j, 0)),
            pl.BlockSpec((1, K, H), lambda b, j: (b, 0, 0)),
            pl.BlockSpec((H, D), lambda b, j: (0, 0)),
        ],
        out_specs=[pl.BlockSpec((1, EB, K), lambda b, j: (b, j, 0))],
        out_shape=[jax.ShapeDtypeStruct((B, EPG, K), jnp.float32)],
        interpret=_INTERPRET,
    )(rl3, edge_relu, ue, W3T)[0]


def _score_body(src_ref, dst_ref, table_ref, re_ref, score_ref):
    b = pl.program_id(0)
    srcl = src_ref[0, 0] - b * NPG
    dstl = dst_ref[0, 0] - b * NPG
    nio = lax.broadcasted_iota(jnp.int32, (EPG, NPG), 1)
    oh_s = (srcl[:, None] == nio).astype(jnp.float32)
    oh_d = (dstl[:, None] == nio).astype(jnp.float32)
    tbl = table_ref[0]
    gs = jnp.dot(oh_s, tbl, preferred_element_type=jnp.float32)
    gd = jnp.dot(oh_d, tbl, preferred_element_type=jnp.float32)
    lane = lax.broadcasted_iota(jnp.int32, (EPG, TW), 1)
    osS = jnp.sum(jnp.where(lane == K, gs, 0.0), axis=1, keepdims=True)
    osD = jnp.sum(jnp.where(lane == K, gd, 0.0), axis=1, keepdims=True)
    prod = gs[:, :K] * gd[:, :K] * re_ref[0] * osS * osD
    score_ref[0] = prod


def _score(src3, dst3, table, re):
    return pl.pallas_call(
        _score_body, grid=(B,),
        in_specs=[
            pl.BlockSpec((1, 1, EPG), lambda b: (b, 0, 0)),
            pl.BlockSpec((1, 1, EPG), lambda b: (b, 0, 0)),
            pl.BlockSpec((1, NPG, TW), lambda b: (b, 0, 0)),
            pl.BlockSpec((1, EPG, K), lambda b: (b, 0, 0)),
        ],
        out_specs=[pl.BlockSpec((1, EPG, K), lambda b: (b, 0, 0))],
        out_shape=[jax.ShapeDtypeStruct((B, EPG, K), jnp.float32)],
        interpret=_INTERPRET,
    )(src3, dst3, table, re)[0]


M = EPG * K


def _topk_body(score_ref, vals_ref, idxs_ref, scratch):
    scratch[...] = score_ref[...]
    iota = lax.broadcasted_iota(jnp.int32, (B, M), 1)
    r128 = lax.broadcasted_iota(jnp.int32, (B, 128), 1)

    def body(r, carry):
        vals, idxs = carry
        v = scratch[...]
        m = jnp.max(v, axis=1, keepdims=True)
        f = jnp.min(jnp.where(v >= m, iota, M), axis=1, keepdims=True)
        scratch[...] = jnp.where(iota == f, -jnp.inf, v)
        vals = jnp.where(r128 == r, m, vals)
        idxs = jnp.where(r128 == r, f, idxs)
        return vals, idxs

    vals, idxs = lax.fori_loop(
        0, TOPX, body,
        (jnp.zeros((B, 128), jnp.float32), jnp.zeros((B, 128), jnp.int32)))
    vals_ref[...] = vals
    idxs_ref[...] = idxs


def _topk(score_flat):
    return pl.pallas_call(
        _topk_body,
        in_specs=[pl.BlockSpec((B, M), lambda: (0, 0))],
        out_specs=[pl.BlockSpec((B, 128), lambda: (0, 0)),
                   pl.BlockSpec((B, 128), lambda: (0, 0))],
        out_shape=[jax.ShapeDtypeStruct((B, 128), jnp.float32),
                   jax.ShapeDtypeStruct((B, 128), jnp.int32)],
        scratch_shapes=[pltpu.VMEM((B, M), jnp.float32)],
        interpret=_INTERPRET,
    )(score_flat)


def _assemble_body(idx_ref, sd_ref, ps_ref, clsf_ref,
                   src_ref, dst_ref, psel_ref, csel_ref):
    f = idx_ref[0]
    rel = f // K
    pred = f - rel * K
    relb = jax.lax.broadcast_in_dim(rel, (EPG, 128), (0, 1))
    eio = lax.broadcasted_iota(jnp.int32, (EPG, 128), 0)
    ohT = (eio == relb).astype(jnp.float32)
    src_sel = jnp.dot(sd_ref[0, 0:1, :], ohT, preferred_element_type=jnp.float32)
    dst_sel = jnp.dot(sd_ref[0, 1:2, :], ohT, preferred_element_type=jnp.float32)
    predb = jax.lax.broadcast_in_dim(pred, (K, 128), (0, 1))
    kio = lax.broadcasted_iota(jnp.int32, (K, 128), 0)
    ohP = (kio == predb).astype(jnp.float32)
    psel = jnp.dot(ps_ref[0], ohP, preferred_element_type=jnp.float32)
    csel = jnp.dot(clsf_ref[0], ohP, preferred_element_type=jnp.float32)
    src_ref[0] = src_sel.astype(jnp.int32)
    dst_ref[0] = dst_sel.astype(jnp.int32)
    psel_ref[0] = psel
    csel_ref[0] = csel.astype(jnp.int32)


def _assemble(idxs3, sdf3, ps3, clsf3):
    return pl.pallas_call(
        _assemble_body, grid=(B,),
        in_specs=[
            pl.BlockSpec((1, 1, 128), lambda b: (b, 0, 0)),
            pl.BlockSpec((1, 2, EPG), lambda b: (b, 0, 0)),
            pl.BlockSpec((1, 1, K), lambda b: (b, 0, 0)),
            pl.BlockSpec((1, 1, K), lambda b: (b, 0, 0)),
        ],
        out_specs=[pl.BlockSpec((1, 1, 128), lambda b: (b, 0, 0))] * 4,
        out_shape=[
            jax.ShapeDtypeStruct((B, 1, 128), jnp.int32),
            jax.ShapeDtypeStruct((B, 1, 128), jnp.int32),
            jax.ShapeDtypeStruct((B, 1, 128), jnp.float32),
            jax.ShapeDtypeStruct((B, 1, 128), jnp.int32),
        ],
        interpret=_INTERPRET,
    )(idxs3, sdf3, ps3, clsf3)


def kernel(object_linear_features, object_conv_features, relation_linear_features,
           object_scores, W1, W2, W3, Wout, batch, relation_indexes):
    del batch
    cv3 = object_conv_features.reshape(N, C, SS)
    ol3 = object_linear_features.reshape(B, NPG, D)
    rl3 = relation_linear_features.reshape(B, EPG, D)
    os3 = object_scores.reshape(B, 1, NPG)
    src3 = relation_indexes[0].reshape(B, 1, EPG)
    dst3 = relation_indexes[1].reshape(B, 1, EPG)
    W1T = jnp.transpose(W1)
    W2T = jnp.transpose(W2)
    W3T = jnp.transpose(W3)
    WoutT = jnp.transpose(Wout)

    pooled, pos, neg = _conv_stats(cv3)
    pooled3 = pooled.reshape(B, NPG, C)
    pos3 = pos.reshape(B, NPG, C)
    neg3 = neg.reshape(B, NPG, C)

    node_relu, gnode = _node_fwd(ol3, pooled3, W1, W2)
    edge_relu, gedge = _edge_fwd(rl3, W3)
    ps, cls, u, ue = _head(gnode, gedge, Wout, WoutT)

    table = _node_rel(ol3, node_relu, pos3, neg3, os3, u, W1T, W2T)
    re = _edge_rel(rl3, edge_relu, ue, W3T)
    score = _score(src3, dst3, table, re)

    vals, idxs = _topk(score.reshape(B, M))

    sdf3 = jnp.stack([src3[:, 0, :], dst3[:, 0, :]], axis=1).astype(jnp.float32)
    ps3 = ps.reshape(B, 1, K)
    clsf3 = cls.astype(jnp.float32).reshape(B, 1, K)
    src_sel, dst_sel, psel, csel = _assemble(idxs.reshape(B, 1, 128),
                                             sdf3, ps3, clsf3)

    relation_scores_sorted = vals[:, :TOPX].reshape(-1)
    relation_indexes_sorted = jnp.stack(
        [src_sel[:, 0, :TOPX].reshape(-1), dst_sel[:, 0, :TOPX].reshape(-1)], axis=0)
    predicate_scores_out = psel[:, 0, :TOPX].reshape(-1)
    predicate_classes_out = csel[:, 0, :TOPX].reshape(-1)
    n_relations = jnp.full((B,), TOPX, dtype=jnp.int32)
    return (relation_scores_sorted, relation_indexes_sorted, predicate_scores_out,
            predicate_classes_out, n_relations)


# speedup vs baseline: 3.3411x; 1.0726x over previous
import functools
import jax
import jax.numpy as jnp
from jax import lax
from jax.experimental import pallas as pl
from jax.experimental.pallas import tpu as pltpu

N = 2000; E = 16000; B = 8; D = 1024; C = 128; S = 7
H = 256; P = 50; K = 10; TOPX = 100
NPG = N // B; EPG = E // B
SS = S * S
EB = 1000
TW = 16

_INTERPRET = False


CSS = C * SS


def _conv_stats_body(cv_ref, pooled_ref, pos_ref, neg_ref, G_ref):
    i = pl.program_id(0)

    @pl.when(i == 0)
    def _():
        j = lax.broadcasted_iota(jnp.int32, (CSS, C), 0)
        c = lax.broadcasted_iota(jnp.int32, (CSS, C), 1)
        G_ref[...] = ((j >= c * SS) & (j < c * SS + SS)).astype(jnp.float32)

    x = cv_ref[...]
    G = G_ref[...]
    sum_x = jnp.dot(x, G, preferred_element_type=jnp.float32)
    posm = jnp.dot(jnp.maximum(x, 0.0), G, preferred_element_type=jnp.float32)
    pooled_ref[...] = sum_x * (1.0 / SS)
    pos_ref[...] = posm
    neg_ref[...] = posm - sum_x


def _conv_stats(cv2):
    BN = 400
    grid = (N // BN,)
    return pl.pallas_call(
        _conv_stats_body, grid=grid,
        in_specs=[pl.BlockSpec((BN, CSS), lambda i: (i, 0))],
        out_specs=[pl.BlockSpec((BN, C), lambda i: (i, 0))] * 3,
        out_shape=[jax.ShapeDtypeStruct((N, C), jnp.float32)] * 3,
        scratch_shapes=[pltpu.VMEM((CSS, C), jnp.float32)],
        interpret=_INTERPRET,
    )(cv2)


def _node_fwd_body(ol_ref, pooled_ref, W1_ref, W2_ref, node_ref, gnode_ref):
    pre = (jnp.dot(ol_ref[0], W1_ref[...], preferred_element_type=jnp.float32)
           + jnp.dot(pooled_ref[0], W2_ref[...], preferred_element_type=jnp.float32))
    nr = jnp.maximum(pre, 0.0)
    node_ref[0] = nr
    gnode_ref[0, 0] = jnp.sum(nr, axis=0)


def _node_fwd(ol3, pooled3, W1, W2):
    return pl.pallas_call(
        _node_fwd_body, grid=(B,),
        in_specs=[
            pl.BlockSpec((1, NPG, D), lambda b: (b, 0, 0)),
            pl.BlockSpec((1, NPG, C), lambda b: (b, 0, 0)),
            pl.BlockSpec((D, H), lambda b: (0, 0)),
            pl.BlockSpec((C, H), lambda b: (0, 0)),
        ],
        out_specs=[
            pl.BlockSpec((1, NPG, H), lambda b: (b, 0, 0)),
            pl.BlockSpec((1, 1, H), lambda b: (b, 0, 0)),
        ],
        out_shape=[
            jax.ShapeDtypeStruct((B, NPG, H), jnp.float32),
            jax.ShapeDtypeStruct((B, 1, H), jnp.float32),
        ],
        interpret=_INTERPRET,
    )(ol3, pooled3, W1, W2)


def _edge_fwd_body(rl_ref, W3_ref, edge_ref, gedge_ref):
    j = pl.program_id(1)
    pre = jnp.dot(rl_ref[0], W3_ref[...], preferred_element_type=jnp.float32)
    nr = jnp.maximum(pre, 0.0)
    edge_ref[0] = nr
    s = jnp.sum(nr, axis=0)

    @pl.when(j == 0)
    def _():
        gedge_ref[0, 0] = s

    @pl.when(j > 0)
    def _():
        gedge_ref[0, 0] += s


def _edge_fwd(rl3, W3):
    return pl.pallas_call(
        _edge_fwd_body, grid=(B, EPG // EB),
        in_specs=[
            pl.BlockSpec((1, EB, D), lambda b, j: (b, j, 0)),
            pl.BlockSpec((D, H), lambda b, j: (0, 0)),
        ],
        out_specs=[
            pl.BlockSpec((1, EB, H), lambda b, j: (b, j, 0)),
            pl.BlockSpec((1, 1, H), lambda b, j: (b, 0, 0)),
        ],
        out_shape=[
            jax.ShapeDtypeStruct((B, EPG, H), jnp.float32),
            jax.ShapeDtypeStruct((B, 1, H), jnp.float32),
        ],
        interpret=_INTERPRET,
    )(rl3, W3)


def _head_body(gn_ref, ge_ref, Wout_ref, WoutT_ref,
               ps_ref, cls_ref, u_ref, ue_ref):
    g = gn_ref[:, 0, :] + ge_ref[:, 0, :]
    logits = jnp.dot(g, Wout_ref[...], preferred_element_type=jnp.float32)
    probs = jax.nn.sigmoid(logits)
    iota = lax.broadcasted_iota(jnp.int32, (B, P), 1)
    v = probs
    vals, clss = [], []
    for _ in range(K):
        m = jnp.max(v, axis=1, keepdims=True)
        cls = jnp.min(jnp.where(v >= m, iota, P), axis=1, keepdims=True)
        vals.append(m)
        clss.append(cls)
        v = jnp.where(iota == cls, -1.0, v)
    ps = jnp.concatenate(vals, axis=1)
    ps_ref[...] = ps
    cls_ref[...] = jnp.concatenate(clss, axis=1)
    us, ues = [], []
    for k in range(K):
        onehot = (iota == clss[k]).astype(jnp.float32)
        sp = vals[k] * (1.0 - vals[k])
        u_k = jnp.dot(onehot * sp, WoutT_ref[...],
                      preferred_element_type=jnp.float32)
        us.append(u_k[:, None, :])
        ues.append((u_k * vals[k])[:, None, :])
    u_ref[...] = jnp.concatenate(us, axis=1)
    ue_ref[...] = jnp.concatenate(ues, axis=1)


def _head(gnode, gedge, Wout, WoutT):
    return pl.pallas_call(
        _head_body,
        in_specs=[
            pl.BlockSpec((B, 1, H), lambda: (0, 0, 0)),
            pl.BlockSpec((B, 1, H), lambda: (0, 0, 0)),
            pl.BlockSpec((H, P), lambda: (0, 0)),
            pl.BlockSpec((P, H), lambda: (0, 0)),
        ],
        out_specs=[
            pl.BlockSpec((B, K), lambda: (0, 0)),
            pl.BlockSpec((B, K), lambda: (0, 0)),
            pl.BlockSpec((B, K, H), lambda: (0, 0, 0)),
            pl.BlockSpec((B, K, H), lambda: (0, 0, 0)),
        ],
        out_shape=[
            jax.ShapeDtypeStruct((B, K), jnp.float32),
            jax.ShapeDtypeStruct((B, K), jnp.int32),
            jax.ShapeDtypeStruct((B, K, H), jnp.float32),
            jax.ShapeDtypeStruct((B, K, H), jnp.float32),
        ],
        interpret=_INTERPRET,
    )(gnode, gedge, Wout, WoutT)


def _node_rel_body(ol_ref, node_ref, pos_ref, neg_ref, os_ref, u_ref,
                   W1T_ref, W2T_ref, table_ref):
    mask = (node_ref[0] > 0.0).astype(jnp.float32)
    ol = ol_ref[0]
    pos = pos_ref[0]
    neg = neg_ref[0]
    lane = lax.broadcasted_iota(jnp.int32, (NPG, TW), 1)
    tbl = jnp.zeros((NPG, TW), jnp.float32)
    for k in range(K):
        mk = mask * u_ref[0, k][None, :]
        gol = jnp.dot(mk, W1T_ref[...], preferred_element_type=jnp.float32)
        rel_lin = jnp.sum(jnp.maximum(ol * gol, 0.0), axis=1)
        gp = jnp.dot(mk, W2T_ref[...], preferred_element_type=jnp.float32) * (1.0 / SS)
        rel_conv = jnp.sum(jnp.maximum(gp, 0.0) * pos
                           + jnp.maximum(-gp, 0.0) * neg, axis=1)
        tbl = jnp.where(lane == k, (rel_lin + rel_conv)[:, None], tbl)
    tbl = jnp.where(lane == K, os_ref[0, 0][:, None], tbl)
    table_ref[0] = tbl


def _node_rel(ol3, node_relu, pos3, neg3, os3, u, W1T, W2T):
    return pl.pallas_call(
        _node_rel_body, grid=(B,),
        in_specs=[
            pl.BlockSpec((1, NPG, D), lambda b: (b, 0, 0)),
            pl.BlockSpec((1, NPG, H), lambda b: (b, 0, 0)),
            pl.BlockSpec((1, NPG, C), lambda b: (b, 0, 0)),
            pl.BlockSpec((1, NPG, C), lambda b: (b, 0, 0)),
            pl.BlockSpec((1, 1, NPG), lambda b: (b, 0, 0)),
            pl.BlockSpec((1, K, H), lambda b: (b, 0, 0)),
            pl.BlockSpec((H, D), lambda b: (0, 0)),
            pl.BlockSpec((H, C), lambda b: (0, 0)),
        ],
        out_specs=[pl.BlockSpec((1, NPG, TW), lambda b: (b, 0, 0))],
        out_shape=[jax.ShapeDtypeStruct((B, NPG, TW), jnp.float32)],
        interpret=_INTERPRET,
    )(ol3, node_relu, pos3, neg3, os3, u, W1T, W2T)[0]


def _edge_rel_body(rl_ref, edge_ref, ue_ref, W3T_ref, re_ref):
    mask = (edge_ref[0] > 0.0).astype(jnp.float32)
    rl = rl_ref[0]
    lane = lax.broadcasted_iota(jnp.int32, (EB, K), 1)
    re = jnp.zeros((EB, K), jnp.float32)
    for k in range(K):
        mk = mask * ue_ref[0, k][None, :]
        grl = jnp.dot(mk, W3T_ref[...], preferred_element_type=jnp.float32)
        re_k = jnp.sum(jnp.maximum(rl * grl, 0.0), axis=1)
        re = jnp.where(lane == k, re_k[:, None], re)
    re_ref[0] = re


def _edge_rel(rl3, edge_relu, ue, W3T):
    return pl.pallas_call(
        _edge_rel_body, grid=(B, EPG // EB),
        in_specs=[
            pl.BlockSpec((1, EB, D), lambda b, j: (b, j, 0)),
            pl.BlockSpec((1, EB, H), lambda b, j: (b, j, 0)),
            pl.BlockSpec((1, K, H), lambda b, j: (b, 0, 0)),
            pl.BlockSpec((H, D), lambda b, j: (0, 0)),
        ],
        out_specs=[pl.BlockSpec((1, EB, K), lambda b, j: (b, j, 0))],
        out_shape=[jax.ShapeDtypeStruct((B, EPG, K), jnp.float32)],
        interpret=_INTERPRET,
    )(rl3, edge_relu, ue, W3T)[0]


def _score_body(src_ref, dst_ref, table_ref, re_ref, score_ref):
    b = pl.program_id(0)
    srcl = src_ref[0, 0] - b * NPG
    dstl = dst_ref[0, 0] - b * NPG
    nio = lax.broadcasted_iota(jnp.int32, (EPG, NPG), 1)
    oh_s = (srcl[:, None] == nio).astype(jnp.float32)
    oh_d = (dstl[:, None] == nio).astype(jnp.float32)
    tbl = table_ref[0]
    gs = jnp.dot(oh_s, tbl, preferred_element_type=jnp.float32)
    gd = jnp.dot(oh_d, tbl, preferred_element_type=jnp.float32)
    lane = lax.broadcasted_iota(jnp.int32, (EPG, TW), 1)
    osS = jnp.sum(jnp.where(lane == K, gs, 0.0), axis=1, keepdims=True)
    osD = jnp.sum(jnp.where(lane == K, gd, 0.0), axis=1, keepdims=True)
    prod = gs[:, :K] * gd[:, :K] * re_ref[0] * osS * osD
    score_ref[0] = prod


def _score(src3, dst3, table, re):
    return pl.pallas_call(
        _score_body, grid=(B,),
        in_specs=[
            pl.BlockSpec((1, 1, EPG), lambda b: (b, 0, 0)),
            pl.BlockSpec((1, 1, EPG), lambda b: (b, 0, 0)),
            pl.BlockSpec((1, NPG, TW), lambda b: (b, 0, 0)),
            pl.BlockSpec((1, EPG, K), lambda b: (b, 0, 0)),
        ],
        out_specs=[pl.BlockSpec((1, EPG, K), lambda b: (b, 0, 0))],
        out_shape=[jax.ShapeDtypeStruct((B, EPG, K), jnp.float32)],
        interpret=_INTERPRET,
    )(src3, dst3, table, re)[0]


M = EPG * K


def _topk_body(score_ref, vals_ref, idxs_ref, scratch):
    scratch[...] = score_ref[...]
    iota = lax.broadcasted_iota(jnp.int32, (B, M), 1)
    r128 = lax.broadcasted_iota(jnp.int32, (B, 128), 1)

    def body(r, carry):
        vals, idxs = carry
        v = scratch[...]
        m = jnp.max(v, axis=1, keepdims=True)
        f = jnp.min(jnp.where(v >= m, iota, M), axis=1, keepdims=True)
        scratch[...] = jnp.where(iota == f, -jnp.inf, v)
        vals = jnp.where(r128 == r, m, vals)
        idxs = jnp.where(r128 == r, f, idxs)
        return vals, idxs

    vals, idxs = lax.fori_loop(
        0, TOPX, body,
        (jnp.zeros((B, 128), jnp.float32), jnp.zeros((B, 128), jnp.int32)))
    vals_ref[...] = vals
    idxs_ref[...] = idxs


def _topk(score_flat):
    return pl.pallas_call(
        _topk_body,
        in_specs=[pl.BlockSpec((B, M), lambda: (0, 0))],
        out_specs=[pl.BlockSpec((B, 128), lambda: (0, 0)),
                   pl.BlockSpec((B, 128), lambda: (0, 0))],
        out_shape=[jax.ShapeDtypeStruct((B, 128), jnp.float32),
                   jax.ShapeDtypeStruct((B, 128), jnp.int32)],
        scratch_shapes=[pltpu.VMEM((B, M), jnp.float32)],
        interpret=_INTERPRET,
    )(score_flat)


def _assemble_body(idx_ref, sd_ref, ps_ref, clsf_ref,
                   src_ref, dst_ref, psel_ref, csel_ref):
    f = idx_ref[0]
    rel = f // K
    pred = f - rel * K
    relb = jax.lax.broadcast_in_dim(rel, (EPG, 128), (0, 1))
    eio = lax.broadcasted_iota(jnp.int32, (EPG, 128), 0)
    ohT = (eio == relb).astype(jnp.float32)
    src_sel = jnp.dot(sd_ref[0, 0:1, :], ohT, preferred_element_type=jnp.float32)
    dst_sel = jnp.dot(sd_ref[0, 1:2, :], ohT, preferred_element_type=jnp.float32)
    predb = jax.lax.broadcast_in_dim(pred, (K, 128), (0, 1))
    kio = lax.broadcasted_iota(jnp.int32, (K, 128), 0)
    ohP = (kio == predb).astype(jnp.float32)
    psel = jnp.dot(ps_ref[0], ohP, preferred_element_type=jnp.float32)
    csel = jnp.dot(clsf_ref[0], ohP, preferred_element_type=jnp.float32)
    src_ref[0] = src_sel.astype(jnp.int32)
    dst_ref[0] = dst_sel.astype(jnp.int32)
    psel_ref[0] = psel
    csel_ref[0] = csel.astype(jnp.int32)


def _assemble(idxs3, sdf3, ps3, clsf3):
    return pl.pallas_call(
        _assemble_body, grid=(B,),
        in_specs=[
            pl.BlockSpec((1, 1, 128), lambda b: (b, 0, 0)),
            pl.BlockSpec((1, 2, EPG), lambda b: (b, 0, 0)),
            pl.BlockSpec((1, 1, K), lambda b: (b, 0, 0)),
            pl.BlockSpec((1, 1, K), lambda b: (b, 0, 0)),
        ],
        out_specs=[pl.BlockSpec((1, 1, 128), lambda b: (b, 0, 0))] * 4,
        out_shape=[
            jax.ShapeDtypeStruct((B, 1, 128), jnp.int32),
            jax.ShapeDtypeStruct((B, 1, 128), jnp.int32),
            jax.ShapeDtypeStruct((B, 1, 128), jnp.float32),
            jax.ShapeDtypeStruct((B, 1, 128), jnp.int32),
        ],
        interpret=_INTERPRET,
    )(idxs3, sdf3, ps3, clsf3)


def kernel(object_linear_features, object_conv_features, relation_linear_features,
           object_scores, W1, W2, W3, Wout, batch, relation_indexes):
    del batch
    cv2 = object_conv_features.reshape(N, CSS)
    ol3 = object_linear_features.reshape(B, NPG, D)
    rl3 = relation_linear_features.reshape(B, EPG, D)
    os3 = object_scores.reshape(B, 1, NPG)
    src3 = relation_indexes[0].reshape(B, 1, EPG)
    dst3 = relation_indexes[1].reshape(B, 1, EPG)
    W1T = jnp.transpose(W1)
    W2T = jnp.transpose(W2)
    W3T = jnp.transpose(W3)
    WoutT = jnp.transpose(Wout)

    pooled, pos, neg = _conv_stats(cv2)
    pooled3 = pooled.reshape(B, NPG, C)
    pos3 = pos.reshape(B, NPG, C)
    neg3 = neg.reshape(B, NPG, C)

    node_relu, gnode = _node_fwd(ol3, pooled3, W1, W2)
    edge_relu, gedge = _edge_fwd(rl3, W3)
    ps, cls, u, ue = _head(gnode, gedge, Wout, WoutT)

    table = _node_rel(ol3, node_relu, pos3, neg3, os3, u, W1T, W2T)
    re = _edge_rel(rl3, edge_relu, ue, W3T)
    score = _score(src3, dst3, table, re)

    vals, idxs = _topk(score.reshape(B, M))

    sdf3 = jnp.stack([src3[:, 0, :], dst3[:, 0, :]], axis=1).astype(jnp.float32)
    ps3 = ps.reshape(B, 1, K)
    clsf3 = cls.astype(jnp.float32).reshape(B, 1, K)
    src_sel, dst_sel, psel, csel = _assemble(idxs.reshape(B, 1, 128),
                                             sdf3, ps3, clsf3)

    relation_scores_sorted = vals[:, :TOPX].reshape(-1)
    relation_indexes_sorted = jnp.stack(
        [src_sel[:, 0, :TOPX].reshape(-1), dst_sel[:, 0, :TOPX].reshape(-1)], axis=0)
    predicate_scores_out = psel[:, 0, :TOPX].reshape(-1)
    predicate_classes_out = csel[:, 0, :TOPX].reshape(-1)
    n_relations = jnp.full((B,), TOPX, dtype=jnp.int32)
    return (relation_scores_sorted, relation_indexes_sorted, predicate_scores_out,
            predicate_classes_out, n_relations)
